# baseline (speedup 1.0000x reference)
import functools

import jax
import jax.numpy as jnp
from jax import lax
from jax.experimental import pallas as pl
from jax.experimental.pallas import tpu as pltpu
from jax.experimental.pallas import tpu_sc as plsc

_B, _S, _D = 4, 4096, 2048


def _build_sc_call():
    mesh = plsc.ScalarSubcoreMesh(axis_name="c", num_cores=1)

    @functools.partial(
        pl.kernel,
        out_type=jax.ShapeDtypeStruct((_B, 1, _D), jnp.float32),
        mesh=mesh,
        scratch_types=[
            pltpu.SMEM((1,), jnp.int32),
        ],
    )
    def gather_kernel(x1_hbm, off_hbm, out_hbm, off_s):
        pltpu.sync_copy(off_hbm, off_s)
        off = off_s[0]
        for b in range(_B):
            pltpu.sync_copy(x1_hbm.at[b, pl.ds(off, 1)], out_hbm.at[b])

    return gather_kernel


_gather = _build_sc_call()


def kernel(x1, x2):
    return _gather(x1, x2.reshape((1,)))


# speedup vs baseline: 1.0658x; 1.0658x over previous
import functools

import jax
import jax.numpy as jnp
from jax import lax
from jax.experimental import pallas as pl
from jax.experimental.pallas import tpu as pltpu
from jax.experimental.pallas import tpu_sc as plsc

_B, _S, _D = 4, 4096, 2048


def _build_sc_call():
    mesh = plsc.VectorSubcoreMesh(
        core_axis_name="c", subcore_axis_name="s", num_cores=1
    )

    @functools.partial(
        pl.kernel,
        out_type=jax.ShapeDtypeStruct((_B, 1, _D), jnp.float32),
        mesh=mesh,
        scratch_types=[
            pltpu.VMEM((16,), jnp.int32),
        ],
    )
    def gather_kernel(x1_hbm, off_hbm, out_hbm, off_v):
        wid = lax.axis_index("s") + lax.axis_index("c")

        @pl.when(wid < _B)
        def _():
            pltpu.sync_copy(off_hbm, off_v.at[pl.ds(0, 1)])
            off = off_v[...][0]
            pltpu.sync_copy(x1_hbm.at[wid, pl.ds(off, 1)], out_hbm.at[wid])

    return gather_kernel


_gather = _build_sc_call()


def kernel(x1, x2):
    return _gather(x1, x2.reshape((1,)))


# speedup vs baseline: 1.1250x; 1.0556x over previous
import functools

import jax
import jax.numpy as jnp
from jax import lax
from jax.experimental import pallas as pl
from jax.experimental.pallas import tpu as pltpu
from jax.experimental.pallas import tpu_sc as plsc

_B, _S, _D = 4, 4096, 2048


def _build_sc_call():
    mesh = plsc.VectorSubcoreMesh(
        core_axis_name="c", subcore_axis_name="s", num_cores=1, num_subcores=4
    )

    @functools.partial(
        pl.kernel,
        out_type=jax.ShapeDtypeStruct((_B, 1, _D), jnp.float32),
        mesh=mesh,
        scratch_types=[
            pltpu.VMEM((16,), jnp.int32),
            pltpu.VMEM((1, _D), jnp.float32),
        ],
    )
    def gather_kernel(x1_hbm, off_hbm, out_hbm, off_v, row_v):
        wid = lax.axis_index("s") + lax.axis_index("c")

        @pl.when(wid < _B)
        def _():
            pltpu.sync_copy(off_hbm, off_v.at[pl.ds(0, 1)])
            off = off_v[...][0]
            pltpu.sync_copy(x1_hbm.at[wid, pl.ds(off, 1)], row_v)
            pltpu.sync_copy(row_v, out_hbm.at[wid])

    return gather_kernel


_gather = _build_sc_call()


def kernel(x1, x2):
    return _gather(x1, x2.reshape((1,)))


# speedup vs baseline: 1.1267x; 1.0016x over previous
import functools

import jax
import jax.numpy as jnp
from jax import lax
from jax.experimental import pallas as pl
from jax.experimental.pallas import tpu as pltpu
from jax.experimental.pallas import tpu_sc as plsc

_B, _S, _D = 4, 4096, 2048


def _build_sc_call():
    mesh = plsc.VectorSubcoreMesh(
        core_axis_name="c", subcore_axis_name="s", num_cores=1, num_subcores=4
    )

    @functools.partial(
        pl.kernel,
        out_type=jax.ShapeDtypeStruct((_B, 1, _D), jnp.float32),
        mesh=mesh,
        scratch_types=[
            pltpu.VMEM((16,), jnp.int32),
            pltpu.VMEM((1, _D), jnp.float32),
        ],
        compiler_params=pltpu.CompilerParams(skip_device_barrier=True),
    )
    def gather_kernel(x1_hbm, off_hbm, out_hbm, off_v, row_v):
        wid = lax.axis_index("s") + lax.axis_index("c")

        @pl.when(wid < _B)
        def _():
            pltpu.sync_copy(off_hbm, off_v.at[pl.ds(0, 1)])
            off = off_v[...][0]
            pltpu.sync_copy(x1_hbm.at[wid, pl.ds(off, 1)], row_v)
            pltpu.sync_copy(row_v, out_hbm.at[wid])

    return gather_kernel


_gather = _build_sc_call()


def kernel(x1, x2):
    return _gather(x1, x2.reshape((1,)))


# speedup vs baseline: 1.1377x; 1.0097x over previous
import functools

import jax
import jax.numpy as jnp
from jax import lax
from jax.experimental import pallas as pl
from jax.experimental.pallas import tpu as pltpu
from jax.experimental.pallas import tpu_sc as plsc

_B, _S, _D = 4, 4096, 2048


def _build_sc_call():
    mesh = plsc.VectorSubcoreMesh(
        core_axis_name="c", subcore_axis_name="s", num_cores=1, num_subcores=4
    )

    @functools.partial(
        pl.kernel,
        out_type=jax.ShapeDtypeStruct((_B, 1, _D), jnp.float32),
        mesh=mesh,
        scratch_types=[
            pltpu.VMEM((16,), jnp.int32),
            pltpu.VMEM((1, _D), jnp.float32),
        ],
    )
    def gather_kernel(x1_hbm, off_hbm, out_hbm, off_v, row_v):
        wid = lax.axis_index("s") + lax.axis_index("c")

        @pl.when(wid < _B)
        def _():
            pltpu.sync_copy(off_hbm, off_v.at[pl.ds(0, 1)])
            off = off_v[...][0]
            pltpu.sync_copy(x1_hbm.at[wid, pl.ds(off, 1)], row_v)
            pltpu.sync_copy(row_v, out_hbm.at[wid])

    return gather_kernel


_gather = _build_sc_call()


def kernel(x1, x2):
    return _gather(x1, x2.reshape((1,)))
